# baseline (device time: 18104 ns/iter reference)
import jax
import jax.numpy as jnp
from jax import lax
from jax.experimental import pallas as pl
from jax.experimental.pallas import tpu as pltpu

N_DEV = 4
N_TOK = 1024
D_IN = 512
D_OUT = 1024
N_EXP = 16
E_LOCAL = 4
CAP = 51
SLOTS = 64
G_SLOTS = E_LOCAL * SLOTS
CHUNK = N_TOK // N_DEV
TILE = 32
MAX_ROWS = 224
N_TILES = MAX_ROWS // TILE


def _cumsum_excl(a):
    n, c = a.shape
    out = jnp.concatenate([jnp.zeros((1, c), a.dtype), a[:-1]], axis=0)
    s = 1
    while s < n:
        out = out + jnp.concatenate(
            [jnp.zeros((s, c), out.dtype), out[:-s]], axis=0
        )
        s *= 2
    return out


def kernel(x, router_W, route_idx, expert_W):
    del router_W

    def body(x_hbm, idx_ref, w_hbm, out_ref,
             x_ref, w_ref, gt_ref, kept_ref, crk_ref, send_ref, recv_ref,
             cnt_ref, base_ref, load_sems, send_sems, recv_sems):
        me = lax.axis_index("i")

        x_copy = pltpu.make_async_copy(x_hbm, x_ref, load_sems.at[0])
        x_copy.start()
        w_copies = []
        for k in range(E_LOCAL):
            wc = pltpu.make_async_copy(
                w_hbm.at[k], w_ref.at[k], load_sems.at[1 + k]
            )
            wc.start()
            w_copies.append(wc)

        recv_ref[...] = jnp.zeros(
            (N_DEV - 1, MAX_ROWS, D_OUT), jnp.bfloat16
        )

        barrier_sem = pltpu.get_barrier_semaphore()
        for k in range(1, N_DEV):
            pl.semaphore_signal(
                barrier_sem, inc=1,
                device_id=((me + k) % N_DEV,),
                device_id_type=pl.DeviceIdType.MESH,
            )
        pl.semaphore_wait(barrier_sem, N_DEV - 1)

        e16 = lax.broadcasted_iota(jnp.int32, (1, N_EXP), 1)
        oh16 = (idx_ref[...] == e16).astype(jnp.float32)
        pos16 = _cumsum_excl(oh16)
        mask16 = oh16 * (pos16 < float(CAP)).astype(jnp.float32)
        kept = jnp.concatenate(
            [
                jnp.sum(mask16[:, 4 * d:4 * d + 4], axis=1, keepdims=True)
                for d in range(N_DEV)
            ],
            axis=1,
        )
        crk = _cumsum_excl(kept)
        kept_ref[...] = kept
        crk_ref[...] = crk

        tok_col = lax.broadcasted_iota(jnp.int32, (N_TOK, 1), 0)
        chunk_ind = (
            (lax.shift_right_logical(tok_col, 8)
             == lax.broadcasted_iota(jnp.int32, (1, N_DEV), 1))
        ).astype(jnp.bfloat16)
        cnt_mat = lax.dot_general(
            chunk_ind, kept.astype(jnp.bfloat16), (((0,), (0,)), ((), ())),
            preferred_element_type=jnp.float32,
        )
        running = [jnp.float32(0.0)] * N_DEV
        for r in range(N_DEV):
            for d in range(N_DEV):
                base_ref[r, d] = running[d]
                c = cnt_mat[r, d]
                cnt_ref[r, d] = c
                running[d] = running[d] + c

        e_col = lax.broadcasted_iota(jnp.int32, (N_EXP, 1), 0)
        sel16 = (
            e_col - E_LOCAL * me
            == lax.broadcasted_iota(jnp.int32, (1, E_LOCAL), 1)
        ).astype(jnp.float32)
        pos4 = jnp.dot(pos16, sel16, preferred_element_type=jnp.float32)
        mask4 = jnp.dot(mask16, sel16, preferred_element_type=jnp.float32)
        slot_i = lax.broadcasted_iota(jnp.int32, (1, SLOTS), 1).astype(
            jnp.float32
        )
        gt_ref[...] = jnp.concatenate(
            [
                (pos4[:, k:k + 1] == slot_i).astype(jnp.float32)
                * mask4[:, k:k + 1]
                for k in range(E_LOCAL)
            ],
            axis=1,
        ).astype(jnp.bfloat16)

        x_copy.wait()
        xg = lax.dot_general(
            gt_ref[...], x_ref[...].astype(jnp.bfloat16),
            (((0,), (0,)), ((), ())), preferred_element_type=jnp.float32,
        ).astype(jnp.bfloat16)

        yg_parts = []
        for k in range(E_LOCAL):
            w_copies[k].wait()
            yg_parts.append(
                jnp.dot(xg[k * SLOTS:(k + 1) * SLOTS, :],
                        w_ref[k].astype(jnp.bfloat16),
                        preferred_element_type=jnp.float32)
            )
        yg = jnp.concatenate(yg_parts, axis=0).astype(jnp.bfloat16)

        sel_me = (
            lax.broadcasted_iota(jnp.int32, (N_DEV, 1), 0) == me
        ).astype(jnp.bfloat16)
        cr_col = jnp.dot(crk.astype(jnp.bfloat16), sel_me,
                         preferred_element_type=jnp.float32)
        ch1_row = (
            lax.shift_right_logical(
                lax.broadcasted_iota(jnp.int32, (1, N_TOK), 1), 8
            )
            + 1
        ).astype(jnp.bfloat16)
        ch1_slot = jnp.dot(ch1_row, gt_ref[...],
                           preferred_element_type=jnp.float32)
        cr_slot = lax.dot_general(
            cr_col.astype(jnp.bfloat16), gt_ref[...], (((0,), (0,)), ((), ())),
            preferred_element_type=jnp.float32,
        )

        m_col = lax.broadcasted_iota(jnp.int32, (MAX_ROWS, 1), 0).astype(
            jnp.float32
        )
        m_row = lax.broadcasted_iota(jnp.int32, (1, MAX_ROWS), 1).astype(
            jnp.float32
        )
        i4 = lax.broadcasted_iota(jnp.int32, (N_DEV, 1), 0)

        rdmas = []
        for k in range(1, N_DEV):
            rr = (me + k) % N_DEV
            base_s = base_ref[rr, me]
            cnt_s = cnt_ref[rr, me]
            p_mat = (
                (m_col == (cr_slot - base_s)).astype(jnp.float32)
                * (ch1_slot == jnp.float32(rr + 1)).astype(jnp.float32)
            ).astype(jnp.bfloat16)
            send_ref[k - 1, :, :] = jnp.dot(
                p_mat, yg, preferred_element_type=jnp.float32
            ).astype(jnp.bfloat16)
            for ti in range(N_TILES):
                rdma = pltpu.make_async_remote_copy(
                    src_ref=send_ref.at[k - 1, pl.ds(ti * TILE, TILE), :],
                    dst_ref=recv_ref.at[k - 1, pl.ds(ti * TILE, TILE), :],
                    send_sem=send_sems.at[k - 1, ti],
                    recv_sem=recv_sems.at[k - 1, ti],
                    device_id=(rr,),
                    device_id_type=pl.DeviceIdType.MESH,
                )
                cond = None if ti == 0 else (jnp.float32(ti * TILE) < cnt_s)
                if cond is None:
                    rdma.start()
                else:
                    @pl.when(cond)
                    def _(rdma=rdma):
                        rdma.start()
                rdmas.append((k, ti, cond, rdma))

        own = jnp.dot(
            gt_ref[pl.ds(me * CHUNK, CHUNK), :], yg,
            preferred_element_type=jnp.float32,
        )

        kept_chunk = kept_ref[pl.ds(me * CHUNK, CHUNK), :].astype(jnp.bfloat16)
        crk_chunk = crk_ref[pl.ds(me * CHUNK, CHUNK), :].astype(jnp.bfloat16)
        for k in range(1, N_DEV):
            dd = (me - k) % N_DEV
            base_v = base_ref[me, dd]
            cnt_v = cnt_ref[me, dd]
            sel_d = (i4 == dd).astype(jnp.bfloat16)
            kb = jnp.dot(kept_chunk, sel_d,
                         preferred_element_type=jnp.float32)
            cb = jnp.dot(crk_chunk, sel_d,
                         preferred_element_type=jnp.float32)
            s_mat = (
                ((cb - base_v) == m_row).astype(jnp.float32) * kb
            ).astype(jnp.bfloat16)
            for ti, cond, rdma in [
                (t, c, r) for (kk, t, c, r) in rdmas if kk == k
            ]:
                if cond is None:
                    rdma.wait_recv()
                else:
                    @pl.when(jnp.float32(ti * TILE) < cnt_v)
                    def _(rdma=rdma):
                        rdma.wait_recv()
            own = own + jnp.dot(
                s_mat, recv_ref[k - 1, :, :],
                preferred_element_type=jnp.float32,
            )

        out_ref[...] = own

        for k, ti, cond, rdma in rdmas:
            if cond is None:
                rdma.wait_send()
            else:
                @pl.when(cond)
                def _(rdma=rdma):
                    rdma.wait_send()

    return pl.pallas_call(
        body,
        out_shape=jax.ShapeDtypeStruct((CHUNK, D_OUT), jnp.float32),
        in_specs=[
            pl.BlockSpec(memory_space=pltpu.MemorySpace.HBM),
            pl.BlockSpec(memory_space=pltpu.VMEM),
            pl.BlockSpec(memory_space=pltpu.MemorySpace.HBM),
        ],
        out_specs=pl.BlockSpec(memory_space=pltpu.VMEM),
        scratch_shapes=[
            pltpu.VMEM((N_TOK, D_IN), jnp.float32),
            pltpu.VMEM((E_LOCAL, D_IN, D_OUT), jnp.float32),
            pltpu.VMEM((N_TOK, G_SLOTS), jnp.bfloat16),
            pltpu.VMEM((N_TOK, N_DEV), jnp.float32),
            pltpu.VMEM((N_TOK, N_DEV), jnp.float32),
            pltpu.VMEM((N_DEV - 1, MAX_ROWS, D_OUT), jnp.bfloat16),
            pltpu.VMEM((N_DEV - 1, MAX_ROWS, D_OUT), jnp.bfloat16),
            pltpu.SMEM((N_DEV, N_DEV), jnp.float32),
            pltpu.SMEM((N_DEV, N_DEV), jnp.float32),
            pltpu.SemaphoreType.DMA((1 + E_LOCAL,)),
            pltpu.SemaphoreType.DMA((N_DEV - 1, N_TILES)),
            pltpu.SemaphoreType.DMA((N_DEV - 1, N_TILES)),
        ],
        compiler_params=pltpu.CompilerParams(collective_id=0),
    )(
        pltpu.with_memory_space_constraint(x, pltpu.MemorySpace.HBM),
        route_idx,
        pltpu.with_memory_space_constraint(expert_W, pltpu.MemorySpace.HBM),
    )


# device time: 16859 ns/iter; 1.0738x vs baseline; 1.0738x over previous
import jax
import jax.numpy as jnp
from jax import lax
from jax.experimental import pallas as pl
from jax.experimental.pallas import tpu as pltpu

N_DEV = 4
N_TOK = 1024
D_IN = 512
D_OUT = 1024
N_EXP = 16
E_LOCAL = 4
CAP = 51
SLOTS = 64
G_SLOTS = E_LOCAL * SLOTS
CHUNK = N_TOK // N_DEV
TILE = 32
MAX_ROWS = 224
N_TILES = MAX_ROWS // TILE


def _cumsum_excl(a):
    n, c = a.shape
    out = jnp.concatenate([jnp.zeros((1, c), a.dtype), a[:-1]], axis=0)
    s = 1
    while s < n:
        out = out + jnp.concatenate(
            [jnp.zeros((s, c), out.dtype), out[:-s]], axis=0
        )
        s *= 2
    return out


def kernel(x, router_W, route_idx, expert_W):
    del router_W

    def body(x_hbm, idx_ref, w_hbm, out_ref,
             x_ref, w_ref, gt_ref, kept_ref, crk_ref, send_ref, recv_ref,
             cnt_ref, base_ref, load_sems, send_sems, recv_sems):
        me = lax.axis_index("i")

        x_copy = pltpu.make_async_copy(x_hbm, x_ref, load_sems.at[0])
        w_copy = pltpu.make_async_copy(w_hbm, w_ref, load_sems.at[1])
        x_copy.start()
        w_copy.start()

        recv_ref[...] = jnp.zeros(
            (N_DEV - 1, MAX_ROWS, D_OUT), jnp.bfloat16
        )

        barrier_sem = pltpu.get_barrier_semaphore()
        for k in range(1, N_DEV):
            pl.semaphore_signal(
                barrier_sem, inc=1,
                device_id=((me + k) % N_DEV,),
                device_id_type=pl.DeviceIdType.MESH,
            )
        pl.semaphore_wait(barrier_sem, N_DEV - 1)

        e16 = lax.broadcasted_iota(jnp.int32, (1, N_EXP), 1)
        oh16 = (idx_ref[...] == e16).astype(jnp.float32)
        pos16 = _cumsum_excl(oh16)
        mask16 = oh16 * (pos16 < float(CAP)).astype(jnp.float32)
        kept = jnp.concatenate(
            [
                jnp.sum(mask16[:, 4 * d:4 * d + 4], axis=1, keepdims=True)
                for d in range(N_DEV)
            ],
            axis=1,
        )
        crk = _cumsum_excl(kept)
        kept_ref[...] = kept
        crk_ref[...] = crk

        tok_col = lax.broadcasted_iota(jnp.int32, (N_TOK, 1), 0)
        chunk_ind = (
            (lax.shift_right_logical(tok_col, 8)
             == lax.broadcasted_iota(jnp.int32, (1, N_DEV), 1))
        ).astype(jnp.bfloat16)
        cnt_mat = lax.dot_general(
            chunk_ind, kept.astype(jnp.bfloat16), (((0,), (0,)), ((), ())),
            preferred_element_type=jnp.float32,
        )
        running = [jnp.float32(0.0)] * N_DEV
        for r in range(N_DEV):
            for d in range(N_DEV):
                base_ref[r, d] = running[d]
                c = cnt_mat[r, d]
                cnt_ref[r, d] = c
                running[d] = running[d] + c

        e_col = lax.broadcasted_iota(jnp.int32, (N_EXP, 1), 0)
        sel16 = (
            e_col - E_LOCAL * me
            == lax.broadcasted_iota(jnp.int32, (1, E_LOCAL), 1)
        ).astype(jnp.float32)
        pos4 = jnp.dot(pos16, sel16, preferred_element_type=jnp.float32)
        mask4 = jnp.dot(mask16, sel16, preferred_element_type=jnp.float32)
        slot_i = lax.broadcasted_iota(jnp.int32, (1, SLOTS), 1).astype(
            jnp.float32
        )
        gt_ref[...] = jnp.concatenate(
            [
                (pos4[:, k:k + 1] == slot_i).astype(jnp.float32)
                * mask4[:, k:k + 1]
                for k in range(E_LOCAL)
            ],
            axis=1,
        ).astype(jnp.bfloat16)

        x_copy.wait()
        xg = lax.dot_general(
            gt_ref[...], x_ref[...].astype(jnp.bfloat16),
            (((0,), (0,)), ((), ())), preferred_element_type=jnp.float32,
        ).astype(jnp.bfloat16)

        w_copy.wait()
        yg = jnp.concatenate(
            [
                jnp.dot(xg[k * SLOTS:(k + 1) * SLOTS, :],
                        w_ref[k].astype(jnp.bfloat16),
                        preferred_element_type=jnp.float32)
                for k in range(E_LOCAL)
            ],
            axis=0,
        ).astype(jnp.bfloat16)

        sel_me = (
            lax.broadcasted_iota(jnp.int32, (N_DEV, 1), 0) == me
        ).astype(jnp.bfloat16)
        cr_col = jnp.dot(crk.astype(jnp.bfloat16), sel_me,
                         preferred_element_type=jnp.float32)
        ch1_row = (
            lax.shift_right_logical(
                lax.broadcasted_iota(jnp.int32, (1, N_TOK), 1), 8
            )
            + 1
        ).astype(jnp.bfloat16)
        ch1_slot = jnp.dot(ch1_row, gt_ref[...],
                           preferred_element_type=jnp.float32)
        cr_slot = lax.dot_general(
            cr_col.astype(jnp.bfloat16), gt_ref[...], (((0,), (0,)), ((), ())),
            preferred_element_type=jnp.float32,
        )

        m_col = lax.broadcasted_iota(jnp.int32, (MAX_ROWS, 1), 0).astype(
            jnp.float32
        )
        m_row = lax.broadcasted_iota(jnp.int32, (1, MAX_ROWS), 1).astype(
            jnp.float32
        )
        i4 = lax.broadcasted_iota(jnp.int32, (N_DEV, 1), 0)

        rdmas = []
        for k in range(1, N_DEV):
            rr = (me + k) % N_DEV
            base_s = base_ref[rr, me]
            cnt_s = cnt_ref[rr, me]
            p_mat = (
                (m_col == (cr_slot - base_s)).astype(jnp.float32)
                * (ch1_slot == jnp.float32(rr + 1)).astype(jnp.float32)
            ).astype(jnp.bfloat16)
            send_ref[k - 1, :, :] = jnp.dot(
                p_mat, yg, preferred_element_type=jnp.float32
            ).astype(jnp.bfloat16)
            for ti in range(N_TILES):
                rdma = pltpu.make_async_remote_copy(
                    src_ref=send_ref.at[k - 1, pl.ds(ti * TILE, TILE), :],
                    dst_ref=recv_ref.at[k - 1, pl.ds(ti * TILE, TILE), :],
                    send_sem=send_sems.at[k - 1, ti],
                    recv_sem=recv_sems.at[k - 1, ti],
                    device_id=(rr,),
                    device_id_type=pl.DeviceIdType.MESH,
                )
                cond = None if ti == 0 else (jnp.float32(ti * TILE) < cnt_s)
                if cond is None:
                    rdma.start()
                else:
                    @pl.when(cond)
                    def _(rdma=rdma):
                        rdma.start()
                rdmas.append((k, ti, cond, rdma))

        own = jnp.dot(
            gt_ref[pl.ds(me * CHUNK, CHUNK), :], yg,
            preferred_element_type=jnp.float32,
        )

        kept_chunk = kept_ref[pl.ds(me * CHUNK, CHUNK), :].astype(jnp.bfloat16)
        crk_chunk = crk_ref[pl.ds(me * CHUNK, CHUNK), :].astype(jnp.bfloat16)
        for k in range(1, N_DEV):
            dd = (me - k) % N_DEV
            base_v = base_ref[me, dd]
            cnt_v = cnt_ref[me, dd]
            sel_d = (i4 == dd).astype(jnp.bfloat16)
            kb = jnp.dot(kept_chunk, sel_d,
                         preferred_element_type=jnp.float32)
            cb = jnp.dot(crk_chunk, sel_d,
                         preferred_element_type=jnp.float32)
            s_mat = (
                ((cb - base_v) == m_row).astype(jnp.float32) * kb
            ).astype(jnp.bfloat16)
            for ti, cond, rdma in [
                (t, c, r) for (kk, t, c, r) in rdmas if kk == k
            ]:
                if cond is None:
                    rdma.wait_recv()
                else:
                    @pl.when(jnp.float32(ti * TILE) < cnt_v)
                    def _(rdma=rdma):
                        rdma.wait_recv()
            own = own + jnp.dot(
                s_mat, recv_ref[k - 1, :, :],
                preferred_element_type=jnp.float32,
            )

        out_ref[...] = own

        for k, ti, cond, rdma in rdmas:
            if cond is None:
                rdma.wait_send()
            else:
                @pl.when(cond)
                def _(rdma=rdma):
                    rdma.wait_send()

    return pl.pallas_call(
        body,
        out_shape=jax.ShapeDtypeStruct((CHUNK, D_OUT), jnp.float32),
        in_specs=[
            pl.BlockSpec(memory_space=pltpu.MemorySpace.HBM),
            pl.BlockSpec(memory_space=pltpu.VMEM),
            pl.BlockSpec(memory_space=pltpu.MemorySpace.HBM),
        ],
        out_specs=pl.BlockSpec(memory_space=pltpu.VMEM),
        scratch_shapes=[
            pltpu.VMEM((N_TOK, D_IN), jnp.float32),
            pltpu.VMEM((E_LOCAL, D_IN, D_OUT), jnp.float32),
            pltpu.VMEM((N_TOK, G_SLOTS), jnp.bfloat16),
            pltpu.VMEM((N_TOK, N_DEV), jnp.float32),
            pltpu.VMEM((N_TOK, N_DEV), jnp.float32),
            pltpu.VMEM((N_DEV - 1, MAX_ROWS, D_OUT), jnp.bfloat16),
            pltpu.VMEM((N_DEV - 1, MAX_ROWS, D_OUT), jnp.bfloat16),
            pltpu.SMEM((N_DEV, N_DEV), jnp.float32),
            pltpu.SMEM((N_DEV, N_DEV), jnp.float32),
            pltpu.SemaphoreType.DMA((2,)),
            pltpu.SemaphoreType.DMA((N_DEV - 1, N_TILES)),
            pltpu.SemaphoreType.DMA((N_DEV - 1, N_TILES)),
        ],
        compiler_params=pltpu.CompilerParams(collective_id=0),
    )(
        pltpu.with_memory_space_constraint(x, pltpu.MemorySpace.HBM),
        route_idx,
        pltpu.with_memory_space_constraint(expert_W, pltpu.MemorySpace.HBM),
    )
